# Initial kernel scaffold; baseline (speedup 1.0000x reference)
import jax
import jax.numpy as jnp
from jax.experimental import pallas as pl


def kernel(src_table, tgt_table, src_indices, tgt_indices):
    raise NotImplementedError("write your pallas kernel here")


# speedup vs baseline: 1.1090x; 1.1090x over previous
import functools

import jax
import jax.numpy as jnp
from jax import lax
from jax.experimental import pallas as pl
from jax.experimental.pallas import tpu as pltpu
from jax.experimental.pallas import tpu_sc as plsc

EMBED = 32
LANES = 16
B = 16384
L = 50

_info = plsc.get_sparse_core_info()
_NC = _info.num_cores
_NS = _info.num_subcores
NW = _NC * _NS

ROWS_TOTAL = B * L
IDX_COLS = 128
IDX_ROWS = ROWS_TOTAL // IDX_COLS
IDXROWS_PER_W = IDX_ROWS // NW
CH_IDX_ROWS = 8
CHUNK = CH_IDX_ROWS * IDX_COLS
N_CHUNKS = IDXROWS_PER_W // CH_IDX_ROWS
N_GROUPS = CHUNK // LANES

_mesh = plsc.VectorSubcoreMesh(core_axis_name="c", subcore_axis_name="s")


def _pad_fix(idx_v, rows_v):
    lanes = lax.iota(jnp.int32, LANES)
    gpr = IDX_COLS // LANES

    def scan_zero(k, acc):
        r = k // gpr
        c = (k % gpr) * LANES
        v = idx_v[r, pl.ds(c, LANES)]
        return acc + jnp.where(v == 0, 1, 0).astype(jnp.int32)

    acc = lax.fori_loop(0, N_GROUPS, scan_zero,
                        jnp.zeros((LANES,), jnp.int32))
    n_pad = acc[0]
    for i in range(1, LANES):
        n_pad = n_pad + acc[i]

    @pl.when(n_pad > 0)
    def _():
        zeros16 = jnp.zeros((LANES,), jnp.float32)

        def fix_group(k, carry):
            r = k // gpr
            c = (k % gpr) * LANES
            v = idx_v[r, pl.ds(c, LANES)]
            z = jnp.where(v == 0, 1, 0).astype(jnp.int32)
            for l in range(LANES):
                row = r * IDX_COLS + c + l

                @pl.when(z[l] > 0)
                def _zero(row=row):
                    rows_v[row, pl.ds(0, LANES)] = zeros16
                    rows_v[row, pl.ds(LANES, LANES)] = zeros16

            return carry

        lax.fori_loop(0, N_GROUPS, fix_group, 0)


@functools.partial(
    pl.kernel,
    mesh=_mesh,
    out_type=[
        jax.ShapeDtypeStruct((ROWS_TOTAL, EMBED), jnp.float32),
        jax.ShapeDtypeStruct((ROWS_TOTAL, EMBED), jnp.float32),
    ],
    scratch_types=[
        pltpu.VMEM((CH_IDX_ROWS, IDX_COLS), jnp.int32),
        pltpu.VMEM((CHUNK, EMBED), jnp.float32),
        pltpu.SemaphoreType.DMA,
    ],
    compiler_params=pltpu.CompilerParams(use_tc_tiling_on_sc=False),
)
def _emb_lookup(src_tab, tgt_tab, src_idx, tgt_idx, src_out, tgt_out,
                idx_v, rows_v, sem):
    wid = lax.axis_index("s") * _NC + lax.axis_index("c")
    base = wid * IDXROWS_PER_W

    for tab, idx_hbm, out_hbm in ((src_tab, src_idx, src_out),
                                  (tgt_tab, tgt_idx, tgt_out)):
        def chunk_body(i, carry, tab=tab, idx_hbm=idx_hbm, out_hbm=out_hbm):
            row0 = base + i * CH_IDX_ROWS
            pltpu.sync_copy(idx_hbm.at[pl.ds(row0, CH_IDX_ROWS)], idx_v)

            def fire(j, c):
                pltpu.async_copy(tab.at[idx_v.at[j]],
                                 rows_v.at[pl.ds(j * IDX_COLS, IDX_COLS)],
                                 sem)
                return c

            lax.fori_loop(0, CH_IDX_ROWS, fire, 0)

            def drain(j, c):
                pltpu.make_async_copy(
                    tab.at[idx_v.at[j]],
                    rows_v.at[pl.ds(j * IDX_COLS, IDX_COLS)], sem).wait()
                return c

            lax.fori_loop(0, CH_IDX_ROWS, drain, 0)

            _pad_fix(idx_v, rows_v)
            pltpu.sync_copy(rows_v, out_hbm.at[pl.ds(row0 * IDX_COLS, CHUNK)])
            return carry

        lax.fori_loop(0, N_CHUNKS, chunk_body, 0)


def kernel(src_table, tgt_table, src_indices, tgt_indices):
    si = src_indices.reshape(-1).astype(jnp.int32).reshape(IDX_ROWS, IDX_COLS)
    ti = tgt_indices.reshape(-1).astype(jnp.int32).reshape(IDX_ROWS, IDX_COLS)
    so, to = _emb_lookup(src_table, tgt_table, si, ti)
    return (so.reshape(B, L, EMBED), to.reshape(B, L, EMBED))
